# baseline (speedup 1.0000x reference)
import jax
import jax.numpy as jnp
from jax.experimental import pallas as pl

_A = 3
_ATTR = 85
_G = 76
_S = _G * _G
_STRIDE = 8.0
_ANCH_W = (116.0, 156.0, 373.0)
_ANCH_H = (90.0, 198.0, 326.0)


def _decode_kernel(x_ref, o_ref):
    a = pl.program_id(1)
    x = x_ref[0, 0]

    sig = 0.5 * jnp.tanh(0.5 * x) + 0.5

    lead2 = jax.lax.broadcasted_iota(jnp.int32, (2, _G, _G), 0)
    gj = jax.lax.broadcasted_iota(jnp.int32, (2, _G, _G), 2).astype(jnp.float32)
    gi = jax.lax.broadcasted_iota(jnp.int32, (2, _G, _G), 1).astype(jnp.float32)
    top = (sig[0:2] + jnp.where(lead2 == 0, gj, gi)) * _STRIDE

    aw = jnp.where(a == 0, _ANCH_W[0], jnp.where(a == 1, _ANCH_W[1], _ANCH_W[2]))
    ah = jnp.where(a == 0, _ANCH_H[0], jnp.where(a == 1, _ANCH_H[1], _ANCH_H[2]))
    mid = jnp.exp(x[2:4]) * jnp.where(lead2 == 0, aw, ah)

    y = jnp.concatenate([top, mid, sig[4:]], axis=0)
    yt = jnp.transpose(y, (1, 2, 0))
    o_ref[0] = yt.reshape(_S, _ATTR)


def kernel(inputs):
    b = inputs.shape[0]
    x = inputs.reshape(b, _A, _ATTR, _G, _G)
    return pl.pallas_call(
        _decode_kernel,
        grid=(b, _A),
        in_specs=[pl.BlockSpec((1, 1, _ATTR, _G, _G), lambda i, j: (i, j, 0, 0, 0))],
        out_specs=pl.BlockSpec((1, _S, _ATTR), lambda i, j: (i, j, 0)),
        out_shape=jax.ShapeDtypeStruct((b, _A * _S, _ATTR), jnp.float32),
    )(x)


# speedup vs baseline: 1.1324x; 1.1324x over previous
import jax
import jax.numpy as jnp
from jax.experimental import pallas as pl

_A = 3
_ATTR = 85
_G = 76
_S = _G * _G
_STRIDE = 8.0
_ANCH_W = (116.0, 156.0, 373.0)
_ANCH_H = (90.0, 198.0, 326.0)


def _decode_kernel(x_ref, o_ref):
    a = pl.program_id(1)
    x = x_ref[0, 0]

    lane = jax.lax.broadcasted_iota(jnp.int32, (_S, _ATTR), 1)
    sub = jax.lax.broadcasted_iota(jnp.int32, (_S, _ATTR), 0)

    sig = 0.5 * jnp.tanh(0.5 * x) + 0.5
    is_wh = (lane == 2) | (lane == 3)
    val = jnp.where(is_wh, jnp.exp(x), sig)

    gx = (sub % _G).astype(jnp.float32)
    gy = (sub // _G).astype(jnp.float32)
    add = jnp.where(lane == 0, gx, jnp.where(lane == 1, gy, 0.0))

    aw = jnp.where(a == 0, _ANCH_W[0], jnp.where(a == 1, _ANCH_W[1], _ANCH_W[2]))
    ah = jnp.where(a == 0, _ANCH_H[0], jnp.where(a == 1, _ANCH_H[1], _ANCH_H[2]))
    mult = jnp.where(lane < 2, _STRIDE, jnp.where(lane == 2, aw, jnp.where(lane == 3, ah, 1.0)))

    o_ref[0] = (val + add) * mult


def kernel(inputs):
    b = inputs.shape[0]
    xt = jnp.transpose(inputs.reshape(b, _A, _ATTR, _S), (0, 1, 3, 2))
    return pl.pallas_call(
        _decode_kernel,
        grid=(b, _A),
        in_specs=[pl.BlockSpec((1, 1, _S, _ATTR), lambda i, j: (i, j, 0, 0))],
        out_specs=pl.BlockSpec((1, _S, _ATTR), lambda i, j: (i, j, 0)),
        out_shape=jax.ShapeDtypeStruct((b, _A * _S, _ATTR), jnp.float32),
    )(xt)


# speedup vs baseline: 1.2244x; 1.0813x over previous
import numpy as np
import jax
import jax.numpy as jnp
from jax.experimental import pallas as pl

_A = 3
_ATTR = 85
_G = 76
_S = _G * _G
_STRIDE = 8.0
_ANCH_W = (116.0, 156.0, 373.0)
_ANCH_H = (90.0, 198.0, 326.0)


def _make_addm():
    p = np.arange(_S)
    addm = np.zeros((_A * _S, _ATTR), dtype=np.float32)
    for a in range(_A):
        addm[a * _S:(a + 1) * _S, 0] = (p % _G) * _STRIDE
        addm[a * _S:(a + 1) * _S, 1] = (p // _G) * _STRIDE
    return addm


def _make_mult():
    m = np.ones((_A, 1, _ATTR), dtype=np.float32)
    m[:, 0, 0:2] = _STRIDE
    for a in range(_A):
        m[a, 0, 2] = _ANCH_W[a]
        m[a, 0, 3] = _ANCH_H[a]
    return m


_ADDM = jnp.asarray(_make_addm())
_MULT = jnp.asarray(_make_mult())


def _decode_kernel(x_ref, addm_ref, mult_ref, o_ref):
    li = jax.lax.broadcasted_iota(jnp.int32, (1, _ATTR), 1)
    is_wh = (li == 2) | (li == 3)

    x = x_ref[0]
    for a in range(_A):
        xa = x[a]
        sig = 0.5 * jnp.tanh(0.5 * xa) + 0.5
        val = jnp.where(is_wh, jnp.exp(xa), sig)
        o_ref[0, a * _S:(a + 1) * _S, :] = (
            val * mult_ref[a] + addm_ref[a * _S:(a + 1) * _S, :]
        )


def kernel(inputs):
    b = inputs.shape[0]
    xt = jnp.transpose(inputs.reshape(b, _A, _ATTR, _S), (0, 1, 3, 2))
    return pl.pallas_call(
        _decode_kernel,
        grid=(b,),
        in_specs=[
            pl.BlockSpec((1, _A, _S, _ATTR), lambda i: (i, 0, 0, 0)),
            pl.BlockSpec((_A * _S, _ATTR), lambda i: (0, 0)),
            pl.BlockSpec((_A, 1, _ATTR), lambda i: (0, 0, 0)),
        ],
        out_specs=pl.BlockSpec((1, _A * _S, _ATTR), lambda i: (i, 0, 0)),
        out_shape=jax.ShapeDtypeStruct((b, _A * _S, _ATTR), jnp.float32),
    )(xt, _ADDM, _MULT)
